# baseline (speedup 1.0000x reference)
import jax
import jax.numpy as jnp
from jax.experimental import pallas as pl
from jax.experimental.pallas import tpu as pltpu

_R = 16
_NEG_INF = float("-inf")


def _topk_idx128(scores, k):
    bsz, n_exp = scores.shape
    col = jax.lax.broadcasted_iota(jnp.int32, (bsz, n_exp), 1)
    col128 = jax.lax.broadcasted_iota(jnp.int32, (bsz, 128), 1)
    run = scores
    idx128 = jnp.zeros((bsz, 128), dtype=jnp.int32)
    for j in range(k):
        m = jnp.max(run, axis=1, keepdims=True)
        cand = jnp.where(run == m, col, n_exp)
        amin = jnp.min(cand, axis=1, keepdims=True)
        run = jnp.where(col == amin, _NEG_INF, run)
        idx128 = jnp.where(col128 == j, amin, idx128)
    return idx128


def _routing_kernel(q_ref, wa_ref, ba_ref, wb_ref, bb_ref, cfs_ref,
                    b_pool_ref, a_hbm, la_ref, bs_ref,
                    idxa_v, idxa_s, idxb_v, idxb_s, idx_sem, gat_sem):
    bsz = q_ref.shape[0]
    d_out = b_pool_ref.shape[1]
    q = q_ref[...]
    s_a = jax.lax.dot_general(q, wa_ref[...], (((1,), (1,)), ((), ())),
                              preferred_element_type=jnp.float32)
    s_a = s_a + ba_ref[...]
    idxa_v[...] = _topk_idx128(s_a, _R)
    cp = pltpu.make_async_copy(idxa_v, idxa_s, idx_sem)
    cp.start()
    cp.wait()
    for b in range(bsz):
        for r in range(_R):
            e = idxa_s[b, r]
            pltpu.make_async_copy(a_hbm.at[e, r, :], la_ref.at[b, r, :],
                                  gat_sem).start()
    for _ in range(bsz * _R):
        pltpu.make_async_copy(a_hbm.at[0, 0, :], la_ref.at[0, 0, :],
                              gat_sem).wait()

    g_cfs = jax.lax.dot_general(q, wb_ref[...], (((1,), (1,)), ((), ())),
                                preferred_element_type=jnp.float32)
    g_cfs = g_cfs + bb_ref[...]
    for r in range(_R):
        g_cfs = g_cfs + jnp.dot(la_ref[:, r, :], cfs_ref[r],
                                preferred_element_type=jnp.float32)
    idxb_v[...] = _topk_idx128(g_cfs, _R)
    cp = pltpu.make_async_copy(idxb_v, idxb_s, idx_sem)
    cp.start()
    cp.wait()
    oc = 256
    lane = jax.lax.broadcasted_iota(jnp.int32, (oc, _R), 1)
    for b in range(bsz):
        for o0 in range(0, d_out, oc):
            acc = jnp.zeros((oc, _R), dtype=jnp.float32)
            for r in range(_R):
                e = idxb_s[b, r]
                chunk = b_pool_ref[e, o0:o0 + oc, :]
                acc = jnp.where(lane == r, chunk, acc)
            bs_ref[b, o0:o0 + oc, :] = acc


def _apply_kernel(x_ref, la_ref, bs_ref, out_ref):
    x = x_ref[0]
    la = la_ref[0]
    bs = bs_ref[0]
    after = jax.lax.dot_general(x, la, (((1,), (1,)), ((), ())),
                                preferred_element_type=jnp.float32)
    out_ref[0] = jax.lax.dot_general(after, bs, (((1,), (1,)), ((), ())),
                                     preferred_element_type=jnp.float32)


def _run(x, query_signal, A_pool, B_pool, W_A, b_A, W_B, b_B, cfs_W,
         interpret=False):
    bsz, seq, d_in = x.shape
    n_exp = A_pool.shape[0]
    d_out = B_pool.shape[1]
    lora_a, lora_b = pl.pallas_call(
        _routing_kernel,
        in_specs=[
            pl.BlockSpec(memory_space=pltpu.MemorySpace.VMEM),
            pl.BlockSpec(memory_space=pltpu.MemorySpace.VMEM),
            pl.BlockSpec(memory_space=pltpu.MemorySpace.VMEM),
            pl.BlockSpec(memory_space=pltpu.MemorySpace.VMEM),
            pl.BlockSpec(memory_space=pltpu.MemorySpace.VMEM),
            pl.BlockSpec(memory_space=pltpu.MemorySpace.VMEM),
            pl.BlockSpec(memory_space=pltpu.MemorySpace.VMEM),
            pl.BlockSpec(memory_space=pl.ANY),
        ],
        out_shape=[
            jax.ShapeDtypeStruct((bsz, _R, d_in), jnp.float32),
            jax.ShapeDtypeStruct((bsz, d_out, _R), jnp.float32),
        ],
        scratch_shapes=[
            pltpu.VMEM((bsz, 128), jnp.int32),
            pltpu.SMEM((bsz, 128), jnp.int32),
            pltpu.VMEM((bsz, 128), jnp.int32),
            pltpu.SMEM((bsz, 128), jnp.int32),
            pltpu.SemaphoreType.DMA,
            pltpu.SemaphoreType.DMA,
        ],
        interpret=interpret,
    )(query_signal, W_A, b_A.reshape(1, n_exp), W_B, b_B.reshape(1, n_exp),
      cfs_W, B_pool, A_pool)

    out = pl.pallas_call(
        _apply_kernel,
        grid=(bsz,),
        in_specs=[
            pl.BlockSpec((1, seq, d_in), lambda b: (b, 0, 0)),
            pl.BlockSpec((1, _R, d_in), lambda b: (b, 0, 0)),
            pl.BlockSpec((1, d_out, _R), lambda b: (b, 0, 0)),
        ],
        out_specs=pl.BlockSpec((1, seq, d_out), lambda b: (b, 0, 0)),
        out_shape=jax.ShapeDtypeStruct((bsz, seq, d_out), jnp.float32),
        compiler_params=pltpu.CompilerParams(
            dimension_semantics=("arbitrary",)),
        interpret=interpret,
    )(x, lora_a, lora_b)
    return out


def kernel(x, query_signal, A_pool, B_pool, W_A, b_A, W_B, b_B, cfs_W):
    return _run(x, query_signal, A_pool, B_pool, W_A, b_A, W_B, b_B, cfs_W)
